# baseline (speedup 1.0000x reference)
import functools

import numpy as np

import jax
import jax.numpy as jnp
from jax import lax
from jax.experimental import pallas as pl
from jax.experimental.pallas import tpu as pltpu
from jax.experimental.pallas import tpu_sc as plsc

HIDDEN = 768
EPS = 1e-12
LANES = 16
NC, NS = 2, 16
NW = NC * NS
TOK = 4 * 4096
TPW = TOK // NW
CH = 64
NCH = TPW // CH
NH = HIDDEN // LANES


def _allreduce_sum(v):
    lane = lax.iota(jnp.int32, LANES)
    for sh in (8, 4, 2, 1):
        perm = lax.bitwise_xor(lane, jnp.int32(sh))
        v = v + v.at[perm].get(mode="promise_in_bounds", unique_indices=True)
    return v


def _rsqrt_vec(v):
    i = lax.bitcast_convert_type(v, jnp.int32)
    i = jnp.int32(0x5F3759DF) - lax.shift_right_arithmetic(i, 1)
    g = lax.bitcast_convert_type(i, jnp.float32)
    for _ in range(3):
        g = g * (1.5 - 0.5 * v * g * g)
    return g


def _emb_body(ids_hbm, table_hbm, tt_hbm, w_hbm, b_hbm, out_hbm,
              idx_v, rows_v, tt_v, w_v, b_v, gsem):
    wid = lax.axis_index("s") * NC + lax.axis_index("c")
    pltpu.sync_copy(ids_hbm.at[wid], idx_v)
    pltpu.sync_copy(tt_hbm.at[0], tt_v)
    pltpu.sync_copy(w_hbm, w_v)
    pltpu.sync_copy(b_hbm, b_v)

    def chunk(c, carry):
        pltpu.async_copy(table_hbm.at[idx_v.at[c]], rows_v, gsem).wait()

        def token(t, carry2):
            s = jnp.zeros((LANES,), jnp.float32)
            ss = jnp.zeros((LANES,), jnp.float32)
            for j in range(NH):
                sl = pl.ds(j * LANES, LANES)
                x = rows_v[t, sl] + tt_v[sl]
                rows_v[t, sl] = x
                s = s + x
                ss = ss + x * x
            mean_v = _allreduce_sum(s) * (1.0 / HIDDEN)
            msq = _allreduce_sum(ss) * (1.0 / HIDDEN)
            var = jnp.maximum(msq - mean_v * mean_v, 0.0) + EPS
            inv = _rsqrt_vec(var)
            for j in range(NH):
                sl = pl.ds(j * LANES, LANES)
                x = rows_v[t, sl]
                rows_v[t, sl] = (x - mean_v) * inv * w_v[sl] + b_v[sl]
            return carry2

        lax.fori_loop(0, CH, token, 0)
        pltpu.sync_copy(rows_v, out_hbm.at[wid, c])
        return carry

    lax.fori_loop(0, NCH, chunk, 0)


_emb = functools.partial(
    pl.kernel,
    mesh=plsc.VectorSubcoreMesh(core_axis_name="c", subcore_axis_name="s"),
    out_type=jax.ShapeDtypeStruct((NW, NCH, CH, HIDDEN), jnp.float32),
    scratch_types=[
        pltpu.VMEM((NCH, CH), jnp.int32),
        pltpu.VMEM((CH, HIDDEN), jnp.float32),
        pltpu.VMEM((HIDDEN,), jnp.float32),
        pltpu.VMEM((HIDDEN,), jnp.float32),
        pltpu.VMEM((HIDDEN,), jnp.float32),
        pltpu.SemaphoreType.DMA,
    ],
)(_emb_body)


@jax.jit
def kernel(input_ids, word_embeddings, token_type_embeddings, ln_weight, ln_bias):
    b, s = input_ids.shape
    ids = input_ids.reshape(NW, NCH, CH).astype(jnp.int32)
    out = _emb(ids, word_embeddings, token_type_embeddings, ln_weight, ln_bias)
    return out.reshape(b, s, HIDDEN)


# speedup vs baseline: 1.4745x; 1.4745x over previous
import functools

import jax
import jax.numpy as jnp
from jax import lax
from jax.experimental import pallas as pl
from jax.experimental.pallas import tpu as pltpu
from jax.experimental.pallas import tpu_sc as plsc

HIDDEN = 768
EPS = 1e-12
LANES = 16
NC, NS = 2, 16
NW = NC * NS
TOK = 4 * 4096
TPW = TOK // NW
CH = 64
NCH = TPW // CH
NH = HIDDEN // LANES


def _allreduce_sum(v):
    lane = lax.iota(jnp.int32, LANES)
    for sh in (8, 4, 2, 1):
        perm = lax.bitwise_xor(lane, jnp.int32(sh))
        v = v + v.at[perm].get(mode="promise_in_bounds", unique_indices=True)
    return v


def _rsqrt_vec(v):
    i = lax.bitcast_convert_type(v, jnp.int32)
    i = jnp.int32(0x5F3759DF) - lax.shift_right_arithmetic(i, 1)
    g = lax.bitcast_convert_type(i, jnp.float32)
    for _ in range(3):
        g = g * (1.5 - 0.5 * v * g * g)
    return g


def _layer_norm_chunk(buf, w_v, b_v):
    def token(t, carry):
        accs = [jnp.zeros((LANES,), jnp.float32) for _ in range(4)]
        sqs = [jnp.zeros((LANES,), jnp.float32) for _ in range(4)]
        for j in range(NH):
            x = buf[t, pl.ds(j * LANES, LANES)]
            accs[j & 3] = accs[j & 3] + x
            sqs[j & 3] = sqs[j & 3] + x * x
        s = (accs[0] + accs[1]) + (accs[2] + accs[3])
        ss = (sqs[0] + sqs[1]) + (sqs[2] + sqs[3])
        mean_v = _allreduce_sum(s) * (1.0 / HIDDEN)
        msq = _allreduce_sum(ss) * (1.0 / HIDDEN)
        var = jnp.maximum(msq - mean_v * mean_v, 0.0) + EPS
        inv = _rsqrt_vec(var)
        shift = mean_v * inv
        for j in range(NH):
            sl = pl.ds(j * LANES, LANES)
            x = buf[t, sl]
            buf[t, sl] = (x * inv - shift) * w_v[sl] + b_v[sl]
        return carry

    lax.fori_loop(0, CH, token, 0)


def _emb_body(ids_hbm, table_hbm, w_hbm, b_hbm, out_hbm,
              idx_v, rows0, rows1, w_v, b_v, g0, g1, s0, s1):
    wid = lax.axis_index("s") * NC + lax.axis_index("c")
    pltpu.sync_copy(ids_hbm.at[wid], idx_v)
    pltpu.sync_copy(w_hbm, w_v)
    pltpu.sync_copy(b_hbm, b_v)
    bufs = (rows0, rows1)
    gsems = (g0, g1)
    ssems = (s0, s1)

    pltpu.async_copy(table_hbm.at[idx_v.at[0]], rows0, g0)

    def super_body(i, carry):
        for b in range(2):
            c = i * 2 + b
            buf = bufs[b]
            pltpu.make_async_copy(table_hbm.at[idx_v.at[0]], buf,
                                  gsems[b]).wait()
            if b == 1:
                pltpu.make_async_copy(bufs[0], out_hbm.at[wid, 0],
                                      ssems[0]).wait()
            else:
                @pl.when(i >= 1)
                def _():
                    pltpu.make_async_copy(bufs[1], out_hbm.at[wid, 0],
                                          ssems[1]).wait()
            if b == 0:
                pltpu.async_copy(table_hbm.at[idx_v.at[c + 1]], bufs[1],
                                 gsems[1])
            else:
                @pl.when(i < NCH // 2 - 1)
                def _():
                    pltpu.async_copy(table_hbm.at[idx_v.at[c + 1]], bufs[0],
                                     gsems[0])
            _layer_norm_chunk(buf, w_v, b_v)
            pltpu.async_copy(buf, out_hbm.at[wid, c], ssems[b])
        return carry

    lax.fori_loop(0, NCH // 2, super_body, 0)
    pltpu.make_async_copy(bufs[1], out_hbm.at[wid, 0], ssems[1]).wait()


_emb = functools.partial(
    pl.kernel,
    mesh=plsc.VectorSubcoreMesh(core_axis_name="c", subcore_axis_name="s"),
    out_type=jax.ShapeDtypeStruct((NW, NCH, CH, HIDDEN), jnp.float32),
    scratch_types=[
        pltpu.VMEM((NCH, CH), jnp.int32),
        pltpu.VMEM((CH, HIDDEN), jnp.float32),
        pltpu.VMEM((CH, HIDDEN), jnp.float32),
        pltpu.VMEM((HIDDEN,), jnp.float32),
        pltpu.VMEM((HIDDEN,), jnp.float32),
        pltpu.SemaphoreType.DMA,
        pltpu.SemaphoreType.DMA,
        pltpu.SemaphoreType.DMA,
        pltpu.SemaphoreType.DMA,
    ],
)(_emb_body)


@jax.jit
def kernel(input_ids, word_embeddings, token_type_embeddings, ln_weight, ln_bias):
    b, s = input_ids.shape
    ids = input_ids.reshape(NW, NCH, CH).astype(jnp.int32)
    table = word_embeddings + token_type_embeddings[0][None, :]
    out = _emb(ids, table, ln_weight, ln_bias)
    return out.reshape(b, s, HIDDEN)
